# baseline (speedup 1.0000x reference)
import jax
import jax.numpy as jnp
from jax.experimental import pallas as pl
from jax.experimental.pallas import tpu as pltpu

B, N, C = 16, 8732, 85
NCLS = C - 4
EPS = 1e-7
BISECT_ITERS = 32
NC_BLK = 768
NSTEPS = -(-N // NC_BLK)
N_PAD = NSTEPS * NC_BLK


def _body(pt_ref, gt_ref, all_ref, conf_ref, loc_ref,
          ent_s, npos_s, possum_s, hubsum_s):
    j = pl.program_id(0)
    p = pt_ref[...]
    g = gt_ref[...]
    lane = jax.lax.broadcasted_iota(jnp.int32, (B, NC_BLK), 1)
    valid = (lane + j * NC_BLK) < N

    pc = p[:NCLS]
    gc = g[:NCLS]
    q = jnp.where(gc > 0.5, pc, 1.0 - pc)
    bce = jnp.log(jnp.maximum(q, EPS))
    ent = -jnp.sum(bce, axis=0)

    pos = g[0] < 0.5
    posv = pos & valid
    ent_s[:, pl.ds(j * NC_BLK, NC_BLK)] = jnp.where(posv | ~valid, 0.0, ent)

    npos_j = jnp.sum(jnp.where(posv, 1.0, 0.0), axis=1, keepdims=True)
    possum_j = jnp.sum(jnp.where(posv, ent, 0.0), axis=1, keepdims=True)

    d = p[NCLS:] - g[NCLS:]
    hub = jnp.maximum(0.5 * d * d, jnp.abs(d) - 0.5)
    hubrow = jnp.sum(hub, axis=0)
    hubsum_j = jnp.sum(jnp.where(posv, hubrow, 0.0), axis=1, keepdims=True)

    @pl.when(j == 0)
    def _():
        npos_s[...] = npos_j
        possum_s[...] = possum_j
        hubsum_s[...] = hubsum_j

    @pl.when(j > 0)
    def _():
        npos_s[...] = npos_s[...] + npos_j
        possum_s[...] = possum_s[...] + possum_j
        hubsum_s[...] = hubsum_s[...] + hubsum_j

    @pl.when(j == NSTEPS - 1)
    def _():
        e = ent_s[...]
        npos = npos_s[...]
        k = 3.0 * npos

        lo = jnp.zeros((B, 1), jnp.float32)
        hi = jnp.max(e, axis=1, keepdims=True)

        def bis(_, carry):
            lo, hi = carry
            mid = 0.5 * (lo + hi)
            cnt = jnp.sum((e > mid).astype(jnp.float32), axis=1, keepdims=True)
            ge = cnt >= k
            return jnp.where(ge, mid, lo), jnp.where(ge, hi, mid)

        lo, hi = jax.lax.fori_loop(0, BISECT_ITERS, bis, (lo, hi))
        t = 0.5 * (lo + hi)
        above = e > t
        cnt_t = jnp.sum(above.astype(jnp.float32), axis=1, keepdims=True)
        s_above = jnp.sum(jnp.where(above, e, 0.0), axis=1, keepdims=True)
        neg_row = s_above + (k - cnt_t) * t

        neg_total = jnp.sum(neg_row)
        npos_total = jnp.sum(npos)
        pos_total = jnp.sum(possum_s[...])
        hub_total = jnp.sum(hubsum_s[...])

        loss_conf = (pos_total + neg_total) / npos_total
        loss_loc = hub_total / (npos_total * 4.0)
        all_ref[...] = jnp.full((1, 1), loss_conf + loss_loc)
        conf_ref[...] = jnp.full((1, 1), loss_conf)
        loc_ref[...] = jnp.full((1, 1), loss_loc)


@jax.jit
def kernel(pred, gt):
    pred_t = jnp.transpose(pred, (2, 0, 1))
    gt_t = jnp.transpose(gt, (2, 0, 1))

    loss_all, loss_conf, loss_loc = pl.pallas_call(
        _body,
        grid=(NSTEPS,),
        in_specs=[
            pl.BlockSpec((C, B, NC_BLK), lambda j: (0, 0, j)),
            pl.BlockSpec((C, B, NC_BLK), lambda j: (0, 0, j)),
        ],
        out_specs=[
            pl.BlockSpec((1, 1), lambda j: (0, 0)),
            pl.BlockSpec((1, 1), lambda j: (0, 0)),
            pl.BlockSpec((1, 1), lambda j: (0, 0)),
        ],
        out_shape=[
            jax.ShapeDtypeStruct((1, 1), jnp.float32),
            jax.ShapeDtypeStruct((1, 1), jnp.float32),
            jax.ShapeDtypeStruct((1, 1), jnp.float32),
        ],
        scratch_shapes=[
            pltpu.VMEM((B, N_PAD), jnp.float32),
            pltpu.VMEM((B, 1), jnp.float32),
            pltpu.VMEM((B, 1), jnp.float32),
            pltpu.VMEM((B, 1), jnp.float32),
        ],
    )(pred_t, gt_t)

    return (loss_all.reshape(()), loss_conf.reshape(()), loss_loc.reshape(()))


# speedup vs baseline: 1.0196x; 1.0196x over previous
import jax
import jax.numpy as jnp
from jax.experimental import pallas as pl
from jax.experimental.pallas import tpu as pltpu

B, N, C = 16, 8732, 85
NCLS = C - 4
EPS = 1e-7
BISECT_ITERS = 32
NC_BLK = 1024
NSTEPS = -(-N // NC_BLK)
N_PAD = NSTEPS * NC_BLK


def _body(pt_ref, gt_ref, all_ref, conf_ref, loc_ref,
          ent_s, npos_s, possum_s, hubsum_s):
    j = pl.program_id(0)
    p = pt_ref[...]
    g = gt_ref[...]
    lane = jax.lax.broadcasted_iota(jnp.int32, (B, NC_BLK), 1)
    valid = (lane + j * NC_BLK) < N

    pc = p[:NCLS]
    gc = g[:NCLS]
    q = jnp.where(gc > 0.5, pc, 1.0 - pc)
    bce = jnp.log(jnp.maximum(q, EPS))
    ent = -jnp.sum(bce, axis=0)

    pos = g[0] < 0.5
    posv = pos & valid
    ent_s[:, pl.ds(j * NC_BLK, NC_BLK)] = jnp.where(posv | ~valid, 0.0, ent)

    npos_j = jnp.sum(jnp.where(posv, 1.0, 0.0), axis=1, keepdims=True)
    possum_j = jnp.sum(jnp.where(posv, ent, 0.0), axis=1, keepdims=True)

    d = p[NCLS:] - g[NCLS:]
    hub = jnp.maximum(0.5 * d * d, jnp.abs(d) - 0.5)
    hubrow = jnp.sum(hub, axis=0)
    hubsum_j = jnp.sum(jnp.where(posv, hubrow, 0.0), axis=1, keepdims=True)

    @pl.when(j == 0)
    def _():
        npos_s[...] = npos_j
        possum_s[...] = possum_j
        hubsum_s[...] = hubsum_j

    @pl.when(j > 0)
    def _():
        npos_s[...] = npos_s[...] + npos_j
        possum_s[...] = possum_s[...] + possum_j
        hubsum_s[...] = hubsum_s[...] + hubsum_j

    @pl.when(j == NSTEPS - 1)
    def _():
        e = ent_s[...]
        npos = npos_s[...]
        k = 3.0 * npos

        lo = jnp.zeros((B, 1), jnp.float32)
        hi = jnp.max(e, axis=1, keepdims=True)

        def bis(_, carry):
            lo, hi = carry
            mid = 0.5 * (lo + hi)
            cnt = jnp.sum((e > mid).astype(jnp.float32), axis=1, keepdims=True)
            ge = cnt >= k
            return jnp.where(ge, mid, lo), jnp.where(ge, hi, mid)

        lo, hi = jax.lax.fori_loop(0, BISECT_ITERS, bis, (lo, hi))
        t = 0.5 * (lo + hi)
        above = e > t
        cnt_t = jnp.sum(above.astype(jnp.float32), axis=1, keepdims=True)
        s_above = jnp.sum(jnp.where(above, e, 0.0), axis=1, keepdims=True)
        neg_row = s_above + (k - cnt_t) * t

        neg_total = jnp.sum(neg_row)
        npos_total = jnp.sum(npos)
        pos_total = jnp.sum(possum_s[...])
        hub_total = jnp.sum(hubsum_s[...])

        loss_conf = (pos_total + neg_total) / npos_total
        loss_loc = hub_total / (npos_total * 4.0)
        all_ref[...] = jnp.full((1, 1), loss_conf + loss_loc)
        conf_ref[...] = jnp.full((1, 1), loss_conf)
        loc_ref[...] = jnp.full((1, 1), loss_loc)


@jax.jit
def kernel(pred, gt):
    pred_t = jnp.transpose(pred, (2, 0, 1))
    gt_t = jnp.transpose(gt, (2, 0, 1))

    loss_all, loss_conf, loss_loc = pl.pallas_call(
        _body,
        grid=(NSTEPS,),
        in_specs=[
            pl.BlockSpec((C, B, NC_BLK), lambda j: (0, 0, j)),
            pl.BlockSpec((C, B, NC_BLK), lambda j: (0, 0, j)),
        ],
        out_specs=[
            pl.BlockSpec((1, 1), lambda j: (0, 0)),
            pl.BlockSpec((1, 1), lambda j: (0, 0)),
            pl.BlockSpec((1, 1), lambda j: (0, 0)),
        ],
        out_shape=[
            jax.ShapeDtypeStruct((1, 1), jnp.float32),
            jax.ShapeDtypeStruct((1, 1), jnp.float32),
            jax.ShapeDtypeStruct((1, 1), jnp.float32),
        ],
        scratch_shapes=[
            pltpu.VMEM((B, N_PAD), jnp.float32),
            pltpu.VMEM((B, 1), jnp.float32),
            pltpu.VMEM((B, 1), jnp.float32),
            pltpu.VMEM((B, 1), jnp.float32),
        ],
    )(pred_t, gt_t)

    return (loss_all.reshape(()), loss_conf.reshape(()), loss_loc.reshape(()))


# speedup vs baseline: 1.0665x; 1.0460x over previous
import jax
import jax.numpy as jnp
from jax.experimental import pallas as pl
from jax.experimental.pallas import tpu as pltpu

B, N, C = 16, 8732, 85
NCLS = C - 4
EPS = 1e-7
BISECT_ITERS = 24
NC_BLK = 1024
NSTEPS = -(-N // NC_BLK)
N_PAD = NSTEPS * NC_BLK


def _body(pt_ref, gt_ref, all_ref, conf_ref, loc_ref,
          ent_s, npos_s, possum_s, hubsum_s):
    j = pl.program_id(0)
    p = pt_ref[...]
    g = gt_ref[...]
    lane = jax.lax.broadcasted_iota(jnp.int32, (B, NC_BLK), 1)
    valid = (lane + j * NC_BLK) < N

    pc = p[:NCLS]
    gc = g[:NCLS]
    q = jnp.where(gc > 0.5, pc, 1.0 - pc)
    bce = jnp.log(jnp.maximum(q, EPS))
    ent = -jnp.sum(bce, axis=0)

    pos = g[0] < 0.5
    posv = pos & valid
    ent_s[:, pl.ds(j * NC_BLK, NC_BLK)] = jnp.where(posv | ~valid, 0.0, ent)

    npos_j = jnp.sum(jnp.where(posv, 1.0, 0.0), axis=1, keepdims=True)
    possum_j = jnp.sum(jnp.where(posv, ent, 0.0), axis=1, keepdims=True)

    d = p[NCLS:] - g[NCLS:]
    hub = jnp.maximum(0.5 * d * d, jnp.abs(d) - 0.5)
    hubrow = jnp.sum(hub, axis=0)
    hubsum_j = jnp.sum(jnp.where(posv, hubrow, 0.0), axis=1, keepdims=True)

    @pl.when(j == 0)
    def _():
        npos_s[...] = npos_j
        possum_s[...] = possum_j
        hubsum_s[...] = hubsum_j

    @pl.when(j > 0)
    def _():
        npos_s[...] = npos_s[...] + npos_j
        possum_s[...] = possum_s[...] + possum_j
        hubsum_s[...] = hubsum_s[...] + hubsum_j

    @pl.when(j == NSTEPS - 1)
    def _():
        e = ent_s[...]
        npos = npos_s[...]
        k = 3.0 * npos

        lo = jnp.zeros((B, 1), jnp.float32)
        hi = jnp.max(e, axis=1, keepdims=True)

        def bis(_, carry):
            lo, hi = carry
            mid = 0.5 * (lo + hi)
            cnt = jnp.sum((e > mid).astype(jnp.float32), axis=1, keepdims=True)
            ge = cnt >= k
            return jnp.where(ge, mid, lo), jnp.where(ge, hi, mid)

        lo, hi = jax.lax.fori_loop(0, BISECT_ITERS, bis, (lo, hi))
        t = 0.5 * (lo + hi)
        above = e > t
        cnt_t = jnp.sum(above.astype(jnp.float32), axis=1, keepdims=True)
        s_above = jnp.sum(jnp.where(above, e, 0.0), axis=1, keepdims=True)
        neg_row = s_above + (k - cnt_t) * t

        neg_total = jnp.sum(neg_row)
        npos_total = jnp.sum(npos)
        pos_total = jnp.sum(possum_s[...])
        hub_total = jnp.sum(hubsum_s[...])

        loss_conf = (pos_total + neg_total) / npos_total
        loss_loc = hub_total / (npos_total * 4.0)
        all_ref[...] = jnp.full((1, 1), loss_conf + loss_loc)
        conf_ref[...] = jnp.full((1, 1), loss_conf)
        loc_ref[...] = jnp.full((1, 1), loss_loc)


@jax.jit
def kernel(pred, gt):
    pred_t = jnp.transpose(pred, (2, 0, 1))
    gt_t = jnp.transpose(gt, (2, 0, 1))

    loss_all, loss_conf, loss_loc = pl.pallas_call(
        _body,
        grid=(NSTEPS,),
        in_specs=[
            pl.BlockSpec((C, B, NC_BLK), lambda j: (0, 0, j)),
            pl.BlockSpec((C, B, NC_BLK), lambda j: (0, 0, j)),
        ],
        out_specs=[
            pl.BlockSpec((1, 1), lambda j: (0, 0)),
            pl.BlockSpec((1, 1), lambda j: (0, 0)),
            pl.BlockSpec((1, 1), lambda j: (0, 0)),
        ],
        out_shape=[
            jax.ShapeDtypeStruct((1, 1), jnp.float32),
            jax.ShapeDtypeStruct((1, 1), jnp.float32),
            jax.ShapeDtypeStruct((1, 1), jnp.float32),
        ],
        scratch_shapes=[
            pltpu.VMEM((B, N_PAD), jnp.float32),
            pltpu.VMEM((B, 1), jnp.float32),
            pltpu.VMEM((B, 1), jnp.float32),
            pltpu.VMEM((B, 1), jnp.float32),
        ],
    )(pred_t, gt_t)

    return (loss_all.reshape(()), loss_conf.reshape(()), loss_loc.reshape(()))
